# baseline (speedup 1.0000x reference)
import jax
import jax.numpy as jnp
from jax.experimental import pallas as pl
from jax.experimental.pallas import tpu as pltpu
from jax.experimental.pallas import tpu_sc as plsc

W = 128
N_WORKERS = 32
LANES = 16


def kernel(x, emb_weight):
    batch, hist = x.shape
    vocab, emb_dim = emb_weight.shape
    n_w = batch // W
    n_tasks = hist * n_w
    per_worker = n_tasks // N_WORKERS

    tw = emb_weight.reshape(vocab // 2, 2 * emb_dim)
    idx_t = x.T.astype(jnp.int32)

    mesh = plsc.VectorSubcoreMesh(
        core_axis_name="core", subcore_axis_name="subcore"
    )

    @pl.kernel(
        out_type=jax.ShapeDtypeStruct((hist, emb_dim, batch), jnp.float32),
        mesh=mesh,
        scratch_types=[
            pltpu.VMEM((W,), jnp.int32),
            pltpu.VMEM((W,), jnp.int32),
            pltpu.VMEM((W,), jnp.int32),
            pltpu.VMEM((W, 2 * emb_dim), jnp.float32),
            pltpu.VMEM((emb_dim, W), jnp.float32),
        ],
        compiler_params=pltpu.CompilerParams(
            use_tc_tiling_on_sc=True, needs_layout_passes=False
        ),
    )
    def gather_kernel(tw_hbm, i_hbm, o_hbm, idx_v, half_v, off_v, buf_v, out_v):
        nc = jax.lax.axis_size("core")
        wid = jax.lax.axis_index("subcore") * nc + jax.lax.axis_index("core")

        @pl.loop(0, per_worker)
        def _(i):
            t = wid * per_worker + i
            h = t // n_w
            w = t - h * n_w
            pltpu.sync_copy(i_hbm.at[h, pl.ds(w * W, W)], idx_v)
            for c in range(W // LANES):
                v = idx_v[pl.ds(c * LANES, LANES)]
                half_v[pl.ds(c * LANES, LANES)] = jax.lax.shift_right_logical(
                    v, 1
                )
                off_v[pl.ds(c * LANES, LANES)] = (v & 1) * emb_dim
            pltpu.sync_copy(tw_hbm.at[half_v], buf_v)

            @pl.loop(0, emb_dim)
            def _(d):
                for c in range(W // LANES):
                    rows = jax.lax.iota(jnp.int32, LANES) + (c * LANES)
                    cols = off_v[pl.ds(c * LANES, LANES)] + d
                    out_v[d, pl.ds(c * LANES, LANES)] = plsc.load_gather(
                        buf_v, [rows, cols]
                    )

            pltpu.sync_copy(out_v, o_hbm.at[h, :, pl.ds(w * W, W)])

    out = gather_kernel(tw, idx_t)
    return out.transpose(2, 0, 1)


# speedup vs baseline: 1.6199x; 1.6199x over previous
import jax
import jax.numpy as jnp
from jax.experimental import pallas as pl
from jax.experimental.pallas import tpu as pltpu
from jax.experimental.pallas import tpu_sc as plsc

W = 128
N_WORKERS = 32
LANES = 16
NCHUNK = W // LANES


def kernel(x, emb_weight):
    batch, hist = x.shape
    vocab, emb_dim = emb_weight.shape
    n_w = batch // W
    n_tasks = hist * n_w
    per_worker = n_tasks // N_WORKERS

    tw = emb_weight.reshape(vocab // 2, 2 * emb_dim)
    idx_t = x.T.astype(jnp.int32)

    mesh = plsc.VectorSubcoreMesh(
        core_axis_name="core", subcore_axis_name="subcore"
    )

    @pl.kernel(
        out_type=jax.ShapeDtypeStruct((hist, emb_dim, batch), jnp.float32),
        mesh=mesh,
        scratch_types=[
            pltpu.VMEM((2, W), jnp.int32),
            pltpu.VMEM((2, W), jnp.int32),
            pltpu.VMEM((2, W), jnp.int32),
            pltpu.VMEM((2, W, 2 * emb_dim), jnp.float32),
            pltpu.VMEM((2, emb_dim, W), jnp.float32),
            pltpu.SemaphoreType.DMA((2,)),
            pltpu.SemaphoreType.DMA((2,)),
            pltpu.SemaphoreType.DMA((2,)),
        ],
        compiler_params=pltpu.CompilerParams(
            use_tc_tiling_on_sc=True, needs_layout_passes=False
        ),
    )
    def gather_kernel(
        tw_hbm, i_hbm, o_hbm,
        idx_v, half_v, off_v, buf_v, out_v,
        isem, gsem, osem,
    ):
        nc = jax.lax.axis_size("core")
        wid = jax.lax.axis_index("subcore") * nc + jax.lax.axis_index("core")
        t0 = wid * per_worker

        def hw(t):
            h = t // n_w
            return h, t - h * n_w

        def idx_copy(t, b):
            h, w = hw(t)
            return pltpu.make_async_copy(
                i_hbm.at[h, pl.ds(w * W, W)], idx_v.at[b], isem.at[b]
            )

        def gather_copy(b):
            return pltpu.make_async_copy(
                tw_hbm.at[half_v.at[b]], buf_v.at[b], gsem.at[b]
            )

        def out_copy(t, b):
            h, w = hw(t)
            return pltpu.make_async_copy(
                out_v.at[b], o_hbm.at[h, :, pl.ds(w * W, W)], osem.at[b]
            )

        def prep(b):
            for c in range(NCHUNK):
                s = pl.ds(c * LANES, LANES)
                v = idx_v[b, s]
                half_v[b, s] = jax.lax.shift_right_logical(v, 1)
                off_v[b, s] = (v & 1) * emb_dim

        jrows = [
            jax.lax.iota(jnp.int32, LANES) + (c * LANES) for c in range(NCHUNK)
        ]

        def transpose(b):
            offs = [off_v[b, pl.ds(c * LANES, LANES)] for c in range(NCHUNK)]

            @pl.loop(0, emb_dim)
            def _(d):
                for c in range(NCHUNK):
                    out_v[b, d, pl.ds(c * LANES, LANES)] = plsc.load_gather(
                        buf_v.at[b], [jrows[c], offs[c] + d]
                    )

        idx_copy(t0, 0).start()
        idx_copy(t0, 0).wait()
        prep(0)
        gather_copy(0).start()
        idx_copy(t0 + 1, 1).start()

        @pl.loop(0, per_worker, step=2)
        def _(i):
            for b in (0, 1):
                nb = 1 - b
                t = t0 + i + b

                @pl.when(i + b + 1 < per_worker)
                def _():
                    idx_copy(t + 1, nb).wait()
                    prep(nb)
                    gather_copy(nb).start()

                @pl.when(i + b + 2 < per_worker)
                def _():
                    idx_copy(t + 2, b).start()

                @pl.when(i + b >= 2)
                def _():
                    out_copy(t - 2, b).wait()

                gather_copy(b).wait()
                transpose(b)
                out_copy(t, b).start()

        out_copy(t0 + per_worker - 2, 0).wait()
        out_copy(t0 + per_worker - 1, 1).wait()

    out = gather_kernel(tw, idx_t)
    return out.transpose(2, 0, 1)
